# baseline (speedup 1.0000x reference)
import functools

import jax
import jax.numpy as jnp
from jax import lax
from jax.experimental import pallas as pl
from jax.experimental.pallas import tpu as pltpu
from jax.experimental.pallas import tpu_sc as plsc

_NBUF = 5
_LOOK = 3


def _pe_add_kernel(n_rows, d, table_len, chunk, num_workers):
    n_chunks = n_rows // (chunk * num_workers)
    assert n_chunks % _NBUF == 0 and chunk % 16 == 0
    mesh = plsc.VectorSubcoreMesh(core_axis_name="c", subcore_axis_name="s", num_cores=1)
    nc = 1

    scratch = (
        [pltpu.VMEM((table_len * d,), jnp.float32)]
        + [pltpu.VMEM((chunk * d,), jnp.float32) for _ in range(_NBUF)]
        + [pltpu.VMEM((chunk,), jnp.int32) for _ in range(_NBUF)]
        + [pltpu.SemaphoreType.DMA for _ in range(3 * _NBUF)]
    )

    @functools.partial(
        pl.kernel,
        mesh=mesh,
        compiler_params=pltpu.CompilerParams(needs_layout_passes=False),
        out_type=jax.ShapeDtypeStruct((n_rows * d,), jnp.float32),
        scratch_types=scratch,
    )
    def k(x_hbm, idx_hbm, pe_hbm, out_hbm, pe_v, *bufs):
        x_v = bufs[:_NBUF]
        idx_v = bufs[_NBUF : 2 * _NBUF]
        sem_x = bufs[2 * _NBUF : 3 * _NBUF]
        sem_i = bufs[3 * _NBUF : 4 * _NBUF]
        sem_o = bufs[4 * _NBUF : 5 * _NBUF]
        wid = lax.axis_index("s") * nc + lax.axis_index("c")
        base = wid * (chunk * n_chunks)
        pltpu.sync_copy(pe_hbm, pe_v)

        def start_in(i, b):
            r0 = base + i * chunk
            pltpu.async_copy(idx_hbm.at[pl.ds(r0, chunk)], idx_v[b], sem_i[b])
            pltpu.async_copy(x_hbm.at[pl.ds(r0 * d, chunk * d)], x_v[b], sem_x[b])

        for i0 in range(_LOOK):
            start_in(i0, i0)

        def outer(g, carry):
            for b in range(_NBUF):
                i = g * _NBUF + b
                b2 = (b + _LOOK) % _NBUF

                @pl.when(i >= _NBUF - _LOOK)
                def _drain_out():
                    pltpu.make_async_copy(
                        x_v[b2], out_hbm.at[pl.ds(base, chunk * d)], sem_o[b2]
                    ).wait()

                @pl.when(i + _LOOK < n_chunks)
                def _prefetch():
                    start_in(i + _LOOK, b2)

                pltpu.make_async_copy(
                    idx_hbm.at[pl.ds(base, chunk)], idx_v[b], sem_i[b]
                ).wait()
                pltpu.make_async_copy(
                    x_hbm.at[pl.ds(base, chunk * d)], x_v[b], sem_x[b]
                ).wait()

                @plsc.parallel_loop(0, chunk // 16, unroll=2)
                def group_body(g2):
                    idxv = idx_v[b][pl.ds(g2 * 16, 16)] * d
                    for l in range(16):
                        a = idxv[l]
                        rd = (g2 * 16 + l) * d
                        for j in range(0, d, 16):
                            v = pe_v[pl.ds(a + j, 16)]
                            plsc.addupdate(x_v[b].at[pl.ds(rd + j, 16)], v)

                r0 = base + i * chunk
                pltpu.async_copy(
                    x_v[b], out_hbm.at[pl.ds(r0 * d, chunk * d)], sem_o[b]
                )
            return carry

        lax.fori_loop(0, n_chunks // _NBUF, outer, None)
        for i0 in range(n_chunks - (_NBUF - _LOOK), n_chunks):
            pltpu.make_async_copy(
                x_v[i0 % _NBUF], out_hbm.at[pl.ds(base, chunk * d)], sem_o[i0 % _NBUF]
            ).wait()

    return k


def kernel(x, indices, pe):
    b, p, d = x.shape
    n = b * p
    max_len = pe.shape[1]
    num_workers = 16
    chunk = 160
    x2 = x.reshape(n * d)
    idx = indices.reshape(n).astype(jnp.int32)
    pe2 = pe.reshape(max_len * d)
    out = _pe_add_kernel(n, d, max_len, chunk, num_workers)(x2, idx, pe2)
    return out.reshape(b, p, d)


# speedup vs baseline: 1.2911x; 1.2911x over previous
import functools

import jax
import jax.numpy as jnp
from jax import lax
from jax.experimental import pallas as pl
from jax.experimental.pallas import tpu as pltpu
from jax.experimental.pallas import tpu_sc as plsc

_NBUF = 4
_LOOK = 2


def _pe_add_kernel(n_rows, d, table_len, chunk, num_workers):
    n_chunks = n_rows // (chunk * num_workers)
    assert n_chunks % _NBUF == 0 and chunk % 16 == 0
    mesh = plsc.VectorSubcoreMesh(core_axis_name="c", subcore_axis_name="s")
    nc = 2

    scratch = (
        [pltpu.VMEM((table_len * d,), jnp.float32)]
        + [pltpu.VMEM((chunk * d,), jnp.float32) for _ in range(_NBUF)]
        + [pltpu.VMEM((chunk,), jnp.int32) for _ in range(_NBUF)]
        + [pltpu.SemaphoreType.DMA for _ in range(3 * _NBUF)]
    )

    @functools.partial(
        pl.kernel,
        mesh=mesh,
        compiler_params=pltpu.CompilerParams(needs_layout_passes=False),
        out_type=jax.ShapeDtypeStruct((n_rows * d,), jnp.float32),
        scratch_types=scratch,
    )
    def k(x_hbm, idx_hbm, pe_hbm, out_hbm, pe_v, *bufs):
        x_v = bufs[:_NBUF]
        idx_v = bufs[_NBUF : 2 * _NBUF]
        sem_x = bufs[2 * _NBUF : 3 * _NBUF]
        sem_i = bufs[3 * _NBUF : 4 * _NBUF]
        sem_o = bufs[4 * _NBUF : 5 * _NBUF]
        wid = lax.axis_index("s") * nc + lax.axis_index("c")
        base = wid * (chunk * n_chunks)
        pltpu.sync_copy(pe_hbm, pe_v)

        def start_in(i, b):
            r0 = base + i * chunk
            pltpu.async_copy(idx_hbm.at[pl.ds(r0, chunk)], idx_v[b], sem_i[b])
            pltpu.async_copy(x_hbm.at[pl.ds(r0 * d, chunk * d)], x_v[b], sem_x[b])

        for i0 in range(_LOOK):
            start_in(i0, i0)

        def outer(g, carry):
            for b in range(_NBUF):
                i = g * _NBUF + b
                b2 = (b + _LOOK) % _NBUF

                @pl.when(i >= _NBUF - _LOOK)
                def _drain_out():
                    pltpu.make_async_copy(
                        x_v[b2], out_hbm.at[pl.ds(base, chunk * d)], sem_o[b2]
                    ).wait()

                @pl.when(i + _LOOK < n_chunks)
                def _prefetch():
                    start_in(i + _LOOK, b2)

                pltpu.make_async_copy(
                    idx_hbm.at[pl.ds(base, chunk)], idx_v[b], sem_i[b]
                ).wait()
                pltpu.make_async_copy(
                    x_hbm.at[pl.ds(base, chunk * d)], x_v[b], sem_x[b]
                ).wait()

                @plsc.parallel_loop(0, chunk // 16, unroll=2)
                def group_body(g2):
                    idxv = idx_v[b][pl.ds(g2 * 16, 16)] * d
                    for l in range(16):
                        a = idxv[l]
                        rd = (g2 * 16 + l) * d
                        for j in range(0, d, 16):
                            v = pe_v[pl.ds(a + j, 16)]
                            plsc.addupdate(x_v[b].at[pl.ds(rd + j, 16)], v)

                r0 = base + i * chunk
                pltpu.async_copy(
                    x_v[b], out_hbm.at[pl.ds(r0 * d, chunk * d)], sem_o[b]
                )
            return carry

        lax.fori_loop(0, n_chunks // _NBUF, outer, None)
        for i0 in range(n_chunks - (_NBUF - _LOOK), n_chunks):
            pltpu.make_async_copy(
                x_v[i0 % _NBUF], out_hbm.at[pl.ds(base, chunk * d)], sem_o[i0 % _NBUF]
            ).wait()

    return k


def kernel(x, indices, pe):
    b, p, d = x.shape
    n = b * p
    max_len = pe.shape[1]
    num_workers = 32
    chunk = 256
    x2 = x.reshape(n * d)
    idx = indices.reshape(n).astype(jnp.int32)
    pe2 = pe.reshape(max_len * d)
    out = _pe_add_kernel(n, d, max_len, chunk, num_workers)(x2, idx, pe2)
    return out.reshape(b, p, d)


# speedup vs baseline: 2.1986x; 1.7029x over previous
import functools

import jax
import jax.numpy as jnp
from jax import lax
from jax.experimental import pallas as pl
from jax.experimental.pallas import tpu as pltpu
from jax.experimental.pallas import tpu_sc as plsc

_NBUF = 4
_LOOK = 2


def _pe_add_kernel(n_rows, d, table_len, chunk, num_workers):
    n_chunks = n_rows // (chunk * num_workers)
    assert n_chunks % _NBUF == 0 and chunk % 16 == 0
    mesh = plsc.VectorSubcoreMesh(core_axis_name="c", subcore_axis_name="s")
    nc = 2

    scratch = (
        [pltpu.VMEM((table_len * d,), jnp.float32)]
        + [pltpu.VMEM((chunk, d), jnp.float32) for _ in range(_NBUF)]
        + [pltpu.VMEM((chunk,), jnp.int32) for _ in range(_NBUF)]
        + [pltpu.SemaphoreType.DMA for _ in range(3 * _NBUF)]
    )

    @functools.partial(
        pl.kernel,
        mesh=mesh,
        compiler_params=pltpu.CompilerParams(needs_layout_passes=False),
        out_type=jax.ShapeDtypeStruct((n_rows, d), jnp.float32),
        scratch_types=scratch,
    )
    def k(x_hbm, idx_hbm, pe_hbm, out_hbm, pe_v, *bufs):
        x_v = bufs[:_NBUF]
        idx_v = bufs[_NBUF : 2 * _NBUF]
        sem_x = bufs[2 * _NBUF : 3 * _NBUF]
        sem_i = bufs[3 * _NBUF : 4 * _NBUF]
        sem_o = bufs[4 * _NBUF : 5 * _NBUF]
        wid = lax.axis_index("s") * nc + lax.axis_index("c")
        base = wid * (chunk * n_chunks)
        pltpu.sync_copy(pe_hbm, pe_v)

        def start_in(i, b):
            r0 = base + i * chunk
            pltpu.async_copy(idx_hbm.at[pl.ds(r0, chunk)], idx_v[b], sem_i[b])
            pltpu.async_copy(x_hbm.at[pl.ds(r0, chunk)], x_v[b], sem_x[b])

        for i0 in range(_LOOK):
            start_in(i0, i0)

        def outer(g, carry):
            for b in range(_NBUF):
                i = g * _NBUF + b
                b2 = (b + _LOOK) % _NBUF

                @pl.when(i >= _NBUF - _LOOK)
                def _drain_out():
                    pltpu.make_async_copy(
                        x_v[b2], out_hbm.at[pl.ds(base, chunk)], sem_o[b2]
                    ).wait()

                @pl.when(i + _LOOK < n_chunks)
                def _prefetch():
                    start_in(i + _LOOK, b2)

                pltpu.make_async_copy(
                    idx_hbm.at[pl.ds(base, chunk)], idx_v[b], sem_i[b]
                ).wait()
                pltpu.make_async_copy(
                    x_hbm.at[pl.ds(base, chunk)], x_v[b], sem_x[b]
                ).wait()

                @plsc.parallel_loop(0, chunk // 16, unroll=2)
                def group_body(g2):
                    idxv = idx_v[b][pl.ds(g2 * 16, 16)] * d
                    for l in range(16):
                        a = idxv[l]
                        r = g2 * 16 + l
                        for j in range(0, d, 16):
                            v = pe_v[pl.ds(a + j, 16)]
                            plsc.addupdate(x_v[b].at[r, pl.ds(j, 16)], v)

                r0 = base + i * chunk
                pltpu.async_copy(
                    x_v[b], out_hbm.at[pl.ds(r0, chunk)], sem_o[b]
                )
            return carry

        lax.fori_loop(0, n_chunks // _NBUF, outer, None)
        for i0 in range(n_chunks - (_NBUF - _LOOK), n_chunks):
            pltpu.make_async_copy(
                x_v[i0 % _NBUF], out_hbm.at[pl.ds(base, chunk)], sem_o[i0 % _NBUF]
            ).wait()

    return k


def kernel(x, indices, pe):
    b, p, d = x.shape
    n = b * p
    max_len = pe.shape[1]
    num_workers = 32
    chunk = 128
    x2 = x.reshape(n, d)
    idx = indices.reshape(n).astype(jnp.int32)
    pe2 = pe.reshape(max_len * d)
    out = _pe_add_kernel(n, d, max_len, chunk, num_workers)(x2, idx, pe2)
    return out.reshape(b, p, d)
